# baseline (speedup 1.0000x reference)
import functools

import jax
import jax.numpy as jnp
from jax import lax
from jax.experimental import pallas as pl
from jax.experimental.pallas import tpu as pltpu
from jax.experimental.pallas import tpu_sc as plsc

T = 2048
D = 768
DFF = 3072
E = 8
BM = 512
NB = 16
NS = NB * BM
NWORK = 32
ROWS_PER_TILE = NS // NWORK
TOK_PER_TILE = T // NWORK


def _router_body(x_ref, wg_ref, p0_ref, p1_ref, w0_ref, w1_ref, blk_ref):
    x = x_ref[...]
    wg = wg_ref[...]
    logits = lax.dot_general(x, wg, (((1,), (1,)), ((), ())),
                             preferred_element_type=jnp.float32)
    eio = lax.broadcasted_iota(jnp.int32, (T, E), 1)
    m0 = jnp.max(logits, axis=1, keepdims=True)
    idx0 = jnp.min(jnp.where(logits == m0, eio, E), axis=1, keepdims=True)
    lmask = jnp.where(eio == idx0, -jnp.inf, logits)
    m1 = jnp.max(lmask, axis=1, keepdims=True)
    idx1 = jnp.min(jnp.where(lmask == m1, eio, E), axis=1, keepdims=True)
    p1 = jnp.exp(m1 - m0)
    w0 = 1.0 / (1.0 + p1)
    w1 = p1 / (1.0 + p1)

    oh0 = (eio == idx0).astype(jnp.float32)
    oh1 = (eio == idx1).astype(jnp.float32)
    cnt = oh0 + oh1

    prior = jnp.zeros((T, E), jnp.float32)
    tio = lax.broadcasted_iota(jnp.int32, (T, BM), 0)
    cio = lax.broadcasted_iota(jnp.int32, (T, BM), 1)
    for s in range(T // BM):
        ls = (cio + s * BM < tio).astype(jnp.float32)
        cs = cnt[s * BM:(s + 1) * BM, :]
        prior = prior + lax.dot_general(
            ls, cs, (((1,), (0,)), ((), ())),
            preferred_element_type=jnp.float32)

    totals = jnp.sum(cnt, axis=0, keepdims=True)
    pc = (((totals.astype(jnp.int32) + BM - 1) // BM) * BM).astype(jnp.float32)
    l8a = lax.broadcasted_iota(jnp.int32, (E, E), 0)
    l8b = lax.broadcasted_iota(jnp.int32, (E, E), 1)
    l8 = (l8a < l8b).astype(jnp.float32)
    start = lax.dot_general(pc, l8, (((1,), (0,)), ((), ())),
                            preferred_element_type=jnp.float32)

    base = start + prior
    p0_ref[...] = jnp.sum(jnp.where(eio == idx0, base, 0.0),
                          axis=1, keepdims=True).astype(jnp.int32)
    p1_ref[...] = jnp.sum(jnp.where(eio == idx1, base, 0.0),
                          axis=1, keepdims=True).astype(jnp.int32)
    w0_ref[...] = w0
    w1_ref[...] = w1

    endblk = ((start + pc) * (1.0 / BM)).astype(jnp.int32)
    bio = lax.broadcasted_iota(jnp.int32, (NB, E), 0)
    ge = (bio >= jnp.broadcast_to(endblk, (NB, E))).astype(jnp.int32)
    bexp = jnp.minimum(jnp.sum(ge, axis=1, keepdims=True), E - 1)
    blk_ref[...] = jnp.broadcast_to(bexp, (NB, E))


def _router(x_flat, wg):
    return pl.pallas_call(
        _router_body,
        out_shape=(
            jax.ShapeDtypeStruct((T, 1), jnp.int32),
            jax.ShapeDtypeStruct((T, 1), jnp.int32),
            jax.ShapeDtypeStruct((T, 1), jnp.float32),
            jax.ShapeDtypeStruct((T, 1), jnp.float32),
            jax.ShapeDtypeStruct((NB, E), jnp.int32),
        ),
    )(x_flat, wg)


def _dispatch_body(pos0_hbm, pos1_hbm, w0_hbm, w1_hbm, x_hbm,
                   xs_hbm, ws_hbm,
                   pos0_v, pos1_v, w0_v, w1_v, sp0_v, sp1_v, row_v, sw_v,
                   w_sh, sem, wsem):
    cid = lax.axis_index("c")
    sid = lax.axis_index("s")
    wid = sid * 2 + cid
    tbase = wid * TOK_PER_TILE

    pltpu.sync_copy(pos0_hbm.at[pl.ds(tbase, TOK_PER_TILE)], pos0_v)
    pltpu.sync_copy(pos1_hbm.at[pl.ds(tbase, TOK_PER_TILE)], pos1_v)
    pltpu.sync_copy(x_hbm.at[pl.ds(tbase, TOK_PER_TILE)], row_v)
    c0 = pltpu.async_copy(row_v, xs_hbm.at[pos0_v], sem)
    c1 = pltpu.async_copy(row_v, xs_hbm.at[pos1_v], sem)

    sbase = sid * (T // 16)
    pltpu.sync_copy(pos0_hbm.at[pl.ds(sbase, T // 16)], sp0_v)
    pltpu.sync_copy(pos1_hbm.at[pl.ds(sbase, T // 16)], sp1_v)
    pltpu.sync_copy(w0_hbm.at[pl.ds(sbase, T // 16)], w0_v)
    pltpu.sync_copy(w1_hbm.at[pl.ds(sbase, T // 16)], w1_v)
    pltpu.sync_copy(w0_v, w_sh.at[sp0_v])
    pltpu.sync_copy(w1_v, w_sh.at[sp1_v])
    plsc.subcore_barrier()
    pltpu.sync_copy(w_sh.at[pl.ds(wid * ROWS_PER_TILE, ROWS_PER_TILE)], sw_v)
    c2 = pltpu.async_copy(sw_v, ws_hbm.at[pl.ds(wid * ROWS_PER_TILE,
                                                ROWS_PER_TILE)], wsem)
    c0.wait()
    c1.wait()
    c2.wait()


def _dispatch(pos0, pos1, w0, w1, x_flat):
    mesh = plsc.VectorSubcoreMesh(core_axis_name="c", subcore_axis_name="s")
    f = functools.partial(
        pl.kernel,
        out_type=(jax.ShapeDtypeStruct((NS, D), jnp.float32),
                  jax.ShapeDtypeStruct((NS,), jnp.float32)),
        mesh=mesh,
        scratch_types=[
            pltpu.VMEM((TOK_PER_TILE,), jnp.int32),
            pltpu.VMEM((TOK_PER_TILE,), jnp.int32),
            pltpu.VMEM((T // 16,), jnp.float32),
            pltpu.VMEM((T // 16,), jnp.float32),
            pltpu.VMEM((T // 16,), jnp.int32),
            pltpu.VMEM((T // 16,), jnp.int32),
            pltpu.VMEM((TOK_PER_TILE, D), jnp.float32),
            pltpu.VMEM((ROWS_PER_TILE,), jnp.float32),
            pltpu.VMEM_SHARED((NS,), jnp.float32),
            pltpu.SemaphoreType.DMA,
            pltpu.SemaphoreType.DMA,
        ],
    )(_dispatch_body)
    return f(pos0, pos1, w0, w1, x_flat)


def _wconv_body(w1_ref, w2_ref, o1_ref, o2_ref):
    o1_ref[...] = w1_ref[...].astype(jnp.bfloat16)
    o2_ref[...] = w2_ref[...].astype(jnp.bfloat16)


def _wconv(w1, w2):
    return pl.pallas_call(
        _wconv_body,
        grid=(E, 2),
        in_specs=[
            pl.BlockSpec((1, DFF // 2, D), lambda e, i: (e, i, 0)),
            pl.BlockSpec((1, D // 2, DFF), lambda e, i: (e, i, 0)),
        ],
        out_specs=[
            pl.BlockSpec((1, DFF // 2, D), lambda e, i: (e, i, 0)),
            pl.BlockSpec((1, D // 2, DFF), lambda e, i: (e, i, 0)),
        ],
        out_shape=(jax.ShapeDtypeStruct((E, DFF, D), jnp.bfloat16),
                   jax.ShapeDtypeStruct((E, D, DFF), jnp.bfloat16)),
    )(w1, w2)


def _gmm_body(be_ref, x_ref, w1_hbm, b1_ref, w2_hbm, b2_ref, ws_ref, o_ref,
              w1_buf, w2_buf, sems, slot_ref):
    b = pl.program_id(0)
    e = be_ref[b]

    def start_copies(expert, slot):
        pltpu.make_async_copy(w1_hbm.at[expert], w1_buf.at[slot],
                              sems.at[slot, 0]).start()
        pltpu.make_async_copy(w2_hbm.at[expert], w2_buf.at[slot],
                              sems.at[slot, 1]).start()

    @pl.when(b == 0)
    def _():
        slot_ref[0] = 0
        start_copies(e, 0)

    nxt = jnp.minimum(b + 1, NB - 1)
    changes = (b + 1 < NB) & (be_ref[nxt] != e)

    @pl.when(changes)
    def _():
        start_copies(be_ref[nxt], slot_ref[0] ^ 1)

    prv = jnp.maximum(b - 1, 0)

    @pl.when((b == 0) | (be_ref[prv] != e))
    def _():
        s = slot_ref[0]
        pltpu.make_async_copy(w1_hbm.at[e], w1_buf.at[s],
                              sems.at[s, 0]).wait()
        pltpu.make_async_copy(w2_hbm.at[e], w2_buf.at[s],
                              sems.at[s, 1]).wait()

    s = slot_ref[0]
    x = x_ref[...].astype(jnp.bfloat16)
    h = lax.dot_general(x, w1_buf[s], (((1,), (1,)), ((), ())),
                        preferred_element_type=jnp.float32)
    h = h + b1_ref[0]
    h = 0.5 * h * (1.0 + lax.erf(h * 0.7071067811865476))
    o = lax.dot_general(h.astype(jnp.bfloat16), w2_buf[s],
                        (((1,), (1,)), ((), ())),
                        preferred_element_type=jnp.float32)
    o_ref[...] = (o + b2_ref[0]) * ws_ref[...]

    @pl.when(changes)
    def _():
        slot_ref[0] = slot_ref[0] ^ 1


def _gmm(be, xs, ws, w1, b1, w2, b2):
    grid_spec = pltpu.PrefetchScalarGridSpec(
        num_scalar_prefetch=1,
        grid=(NB,),
        in_specs=[
            pl.BlockSpec((BM, D), lambda b, be: (b, 0)),
            pl.BlockSpec(memory_space=pl.ANY),
            pl.BlockSpec((1, 1, DFF), lambda b, be: (be[b], 0, 0)),
            pl.BlockSpec(memory_space=pl.ANY),
            pl.BlockSpec((1, 1, D), lambda b, be: (be[b], 0, 0)),
            pl.BlockSpec((BM, 1), lambda b, be: (b, 0)),
        ],
        out_specs=pl.BlockSpec((BM, D), lambda b, be: (b, 0)),
        scratch_shapes=[
            pltpu.VMEM((2, DFF, D), jnp.bfloat16),
            pltpu.VMEM((2, D, DFF), jnp.bfloat16),
            pltpu.SemaphoreType.DMA((2, 2)),
            pltpu.SMEM((1,), jnp.int32),
        ],
    )
    return pl.pallas_call(
        _gmm_body,
        grid_spec=grid_spec,
        out_shape=jax.ShapeDtypeStruct((NS, D), jnp.float32),
    )(be, xs, w1, b1.reshape(E, 1, DFF), w2, b2.reshape(E, 1, D),
      ws.reshape(NS, 1))


def _combine_body(y_hbm, pos0_hbm, pos1_hbm, out_hbm,
                  p0v, p1v, buf0, buf1, sem):
    cid = lax.axis_index("c")
    sid = lax.axis_index("s")
    wid = sid * 2 + cid
    base = wid * TOK_PER_TILE

    pltpu.sync_copy(pos0_hbm.at[pl.ds(base, TOK_PER_TILE)], p0v)
    pltpu.sync_copy(pos1_hbm.at[pl.ds(base, TOK_PER_TILE)], p1v)
    c0 = pltpu.async_copy(y_hbm.at[p0v], buf0, sem)
    c1 = pltpu.async_copy(y_hbm.at[p1v], buf1, sem)
    c0.wait()
    c1.wait()

    def tbody(t, _):
        def jbody(j, _):
            s = pl.ds(j * 16, 16)
            buf0[t, s] = buf0[t, s] + buf1[t, s]
            return 0
        return lax.fori_loop(0, D // 16, jbody, 0)

    lax.fori_loop(0, TOK_PER_TILE, tbody, 0)
    pltpu.sync_copy(buf0, out_hbm.at[pl.ds(base, TOK_PER_TILE)])


def _combine(y, pos0, pos1):
    mesh = plsc.VectorSubcoreMesh(core_axis_name="c", subcore_axis_name="s")
    f = functools.partial(
        pl.kernel,
        out_type=jax.ShapeDtypeStruct((T, D), jnp.float32),
        mesh=mesh,
        scratch_types=[
            pltpu.VMEM((TOK_PER_TILE,), jnp.int32),
            pltpu.VMEM((TOK_PER_TILE,), jnp.int32),
            pltpu.VMEM((TOK_PER_TILE, D), jnp.float32),
            pltpu.VMEM((TOK_PER_TILE, D), jnp.float32),
            pltpu.SemaphoreType.DMA,
        ],
    )(_combine_body)
    return f(y, pos0, pos1)


def kernel(x, Wg, W1, b1, W2, b2):
    B, S, d = x.shape
    x_flat = x.reshape(T, D)
    p0, p1, w0, w1, blk = _router(x_flat, Wg)
    pos0 = p0.reshape(T)
    pos1 = p1.reshape(T)
    be = blk[:, 0] + 0
    w1_bf, w2_bf = _wconv(W1, W2)
    xs, ws = _dispatch(pos0, pos1, w0.reshape(T), w1.reshape(T), x_flat)
    y = _gmm(be, xs, ws, w1_bf, b1, w2_bf, b2)
    out = _combine(y, pos0, pos1)
    return out.reshape(B, S, D), 0.0


# speedup vs baseline: 1.2214x; 1.2214x over previous
import functools

import jax
import jax.numpy as jnp
from jax import lax
from jax.experimental import pallas as pl
from jax.experimental.pallas import tpu as pltpu
from jax.experimental.pallas import tpu_sc as plsc

T = 2048
D = 768
DFF = 3072
E = 8
BM = 256
NB = 24
NS = NB * BM
NWORK = 32
ROWS_PER_TILE = NS // NWORK
TOK_PER_TILE = T // NWORK


def _router_body(x_ref, wg_ref, p0_ref, p1_ref, w0_ref, w1_ref, blk_ref):
    x = x_ref[...]
    wg = wg_ref[...]
    logits = lax.dot_general(x, wg, (((1,), (1,)), ((), ())),
                             preferred_element_type=jnp.float32)
    eio = lax.broadcasted_iota(jnp.int32, (T, E), 1)
    m0 = jnp.max(logits, axis=1, keepdims=True)
    idx0 = jnp.min(jnp.where(logits == m0, eio, E), axis=1, keepdims=True)
    lmask = jnp.where(eio == idx0, -jnp.inf, logits)
    m1 = jnp.max(lmask, axis=1, keepdims=True)
    idx1 = jnp.min(jnp.where(lmask == m1, eio, E), axis=1, keepdims=True)
    p1 = jnp.exp(m1 - m0)
    w0 = 1.0 / (1.0 + p1)
    w1 = p1 / (1.0 + p1)

    oh0 = (eio == idx0).astype(jnp.float32)
    oh1 = (eio == idx1).astype(jnp.float32)
    cnt = oh0 + oh1

    prior = jnp.zeros((T, E), jnp.float32)
    tio = lax.broadcasted_iota(jnp.int32, (T, BM), 0)
    cio = lax.broadcasted_iota(jnp.int32, (T, BM), 1)
    for s in range(T // BM):
        ls = (cio + s * BM < tio).astype(jnp.float32)
        cs = cnt[s * BM:(s + 1) * BM, :]
        prior = prior + lax.dot_general(
            ls, cs, (((1,), (0,)), ((), ())),
            preferred_element_type=jnp.float32)

    totals = jnp.sum(cnt, axis=0, keepdims=True)
    pc = (((totals.astype(jnp.int32) + BM - 1) // BM) * BM).astype(jnp.float32)
    l8a = lax.broadcasted_iota(jnp.int32, (E, E), 0)
    l8b = lax.broadcasted_iota(jnp.int32, (E, E), 1)
    l8 = (l8a < l8b).astype(jnp.float32)
    start = lax.dot_general(pc, l8, (((1,), (0,)), ((), ())),
                            preferred_element_type=jnp.float32)

    base = start + prior
    p0_ref[...] = jnp.sum(jnp.where(eio == idx0, base, 0.0),
                          axis=1, keepdims=True).astype(jnp.int32)
    p1_ref[...] = jnp.sum(jnp.where(eio == idx1, base, 0.0),
                          axis=1, keepdims=True).astype(jnp.int32)
    w0_ref[...] = w0
    w1_ref[...] = w1

    endblk = ((start + pc) * (1.0 / BM)).astype(jnp.int32)
    bio = lax.broadcasted_iota(jnp.int32, (NB, E), 0)
    ge = (bio >= jnp.broadcast_to(endblk, (NB, E))).astype(jnp.int32)
    bexp = jnp.minimum(jnp.sum(ge, axis=1, keepdims=True), E - 1)
    blk_ref[...] = jnp.broadcast_to(bexp, (NB, E))


def _router(x_flat, wg):
    return pl.pallas_call(
        _router_body,
        out_shape=(
            jax.ShapeDtypeStruct((T, 1), jnp.int32),
            jax.ShapeDtypeStruct((T, 1), jnp.int32),
            jax.ShapeDtypeStruct((T, 1), jnp.float32),
            jax.ShapeDtypeStruct((T, 1), jnp.float32),
            jax.ShapeDtypeStruct((NB, E), jnp.int32),
        ),
    )(x_flat, wg)


def _dispatch_body(pos0_hbm, pos1_hbm, w0_hbm, w1_hbm, x_hbm,
                   xs_hbm, ws_hbm,
                   pos0_v, pos1_v, w0_v, w1_v, sp0_v, sp1_v, row_v, sw_v,
                   w_sh, sem, wsem):
    cid = lax.axis_index("c")
    sid = lax.axis_index("s")
    wid = sid * 2 + cid
    tbase = wid * TOK_PER_TILE

    pltpu.sync_copy(pos0_hbm.at[pl.ds(tbase, TOK_PER_TILE)], pos0_v)
    pltpu.sync_copy(pos1_hbm.at[pl.ds(tbase, TOK_PER_TILE)], pos1_v)
    pltpu.sync_copy(x_hbm.at[pl.ds(tbase, TOK_PER_TILE)], row_v)
    c0 = pltpu.async_copy(row_v, xs_hbm.at[pos0_v], sem)
    c1 = pltpu.async_copy(row_v, xs_hbm.at[pos1_v], sem)

    sbase = sid * (T // 16)
    pltpu.sync_copy(pos0_hbm.at[pl.ds(sbase, T // 16)], sp0_v)
    pltpu.sync_copy(pos1_hbm.at[pl.ds(sbase, T // 16)], sp1_v)
    pltpu.sync_copy(w0_hbm.at[pl.ds(sbase, T // 16)], w0_v)
    pltpu.sync_copy(w1_hbm.at[pl.ds(sbase, T // 16)], w1_v)
    pltpu.sync_copy(w0_v, w_sh.at[sp0_v])
    pltpu.sync_copy(w1_v, w_sh.at[sp1_v])
    plsc.subcore_barrier()
    pltpu.sync_copy(w_sh.at[pl.ds(wid * ROWS_PER_TILE, ROWS_PER_TILE)], sw_v)
    c2 = pltpu.async_copy(sw_v, ws_hbm.at[pl.ds(wid * ROWS_PER_TILE,
                                                ROWS_PER_TILE)], wsem)
    c0.wait()
    c1.wait()
    c2.wait()


def _dispatch(pos0, pos1, w0, w1, x_flat):
    mesh = plsc.VectorSubcoreMesh(core_axis_name="c", subcore_axis_name="s")
    f = functools.partial(
        pl.kernel,
        out_type=(jax.ShapeDtypeStruct((NS, D), jnp.float32),
                  jax.ShapeDtypeStruct((NS,), jnp.float32)),
        mesh=mesh,
        scratch_types=[
            pltpu.VMEM((TOK_PER_TILE,), jnp.int32),
            pltpu.VMEM((TOK_PER_TILE,), jnp.int32),
            pltpu.VMEM((T // 16,), jnp.float32),
            pltpu.VMEM((T // 16,), jnp.float32),
            pltpu.VMEM((T // 16,), jnp.int32),
            pltpu.VMEM((T // 16,), jnp.int32),
            pltpu.VMEM((TOK_PER_TILE, D), jnp.float32),
            pltpu.VMEM((ROWS_PER_TILE,), jnp.float32),
            pltpu.VMEM_SHARED((NS,), jnp.float32),
            pltpu.SemaphoreType.DMA,
            pltpu.SemaphoreType.DMA,
        ],
    )(_dispatch_body)
    return f(pos0, pos1, w0, w1, x_flat)


def _wconv_body(w1_ref, w2_ref, o1_ref, o2_ref):
    o1_ref[...] = w1_ref[...].astype(jnp.bfloat16)
    o2_ref[...] = w2_ref[...].astype(jnp.bfloat16)


def _wconv(w1, w2):
    return pl.pallas_call(
        _wconv_body,
        grid=(E, 2),
        in_specs=[
            pl.BlockSpec((1, DFF // 2, D), lambda e, i: (e, i, 0)),
            pl.BlockSpec((1, D // 2, DFF), lambda e, i: (e, i, 0)),
        ],
        out_specs=[
            pl.BlockSpec((1, DFF // 2, D), lambda e, i: (e, i, 0)),
            pl.BlockSpec((1, D // 2, DFF), lambda e, i: (e, i, 0)),
        ],
        out_shape=(jax.ShapeDtypeStruct((E, DFF, D), jnp.bfloat16),
                   jax.ShapeDtypeStruct((E, D, DFF), jnp.bfloat16)),
    )(w1, w2)


def _gmm_body(be_ref, x_ref, w1_hbm, b1_ref, w2_hbm, b2_ref, ws_ref, o_ref,
              w1_raw, w2_raw, w1_bf, w2_bf, sems, slot_ref):
    b = pl.program_id(0)
    e = be_ref[b]

    def start_copies(expert, slot):
        pltpu.make_async_copy(w1_hbm.at[expert], w1_raw.at[slot],
                              sems.at[slot, 0]).start()
        pltpu.make_async_copy(w2_hbm.at[expert], w2_raw.at[slot],
                              sems.at[slot, 1]).start()

    @pl.when(b == 0)
    def _():
        slot_ref[0] = 0
        start_copies(e, 0)

    nxt = jnp.minimum(b + 1, NB - 1)
    changes = (b + 1 < NB) & (be_ref[nxt] != e)

    @pl.when(changes)
    def _():
        start_copies(be_ref[nxt], slot_ref[0] ^ 1)

    prv = jnp.maximum(b - 1, 0)

    @pl.when((b == 0) | (be_ref[prv] != e))
    def _():
        s = slot_ref[0]
        pltpu.make_async_copy(w1_hbm.at[e], w1_raw.at[s],
                              sems.at[s, 0]).wait()
        pltpu.make_async_copy(w2_hbm.at[e], w2_raw.at[s],
                              sems.at[s, 1]).wait()
        w1_bf[...] = w1_raw[s].astype(jnp.bfloat16)
        w2_bf[...] = w2_raw[s].astype(jnp.bfloat16)

    x = x_ref[...].astype(jnp.bfloat16)
    h = lax.dot_general(x, w1_bf[...], (((1,), (1,)), ((), ())),
                        preferred_element_type=jnp.float32)
    h = h + b1_ref[0]
    h = 0.5 * h * (1.0 + lax.erf(h * 0.7071067811865476))
    o = lax.dot_general(h.astype(jnp.bfloat16), w2_bf[...],
                        (((1,), (1,)), ((), ())),
                        preferred_element_type=jnp.float32)
    o_ref[...] = (o + b2_ref[0]) * ws_ref[...]

    @pl.when(changes)
    def _():
        slot_ref[0] = slot_ref[0] ^ 1


def _gmm(be, xs, ws, w1, b1, w2, b2):
    grid_spec = pltpu.PrefetchScalarGridSpec(
        num_scalar_prefetch=1,
        grid=(NB,),
        in_specs=[
            pl.BlockSpec((BM, D), lambda b, be: (b, 0)),
            pl.BlockSpec(memory_space=pl.ANY),
            pl.BlockSpec((1, 1, DFF), lambda b, be: (be[b], 0, 0)),
            pl.BlockSpec(memory_space=pl.ANY),
            pl.BlockSpec((1, 1, D), lambda b, be: (be[b], 0, 0)),
            pl.BlockSpec((BM, 1), lambda b, be: (b, 0)),
        ],
        out_specs=pl.BlockSpec((BM, D), lambda b, be: (b, 0)),
        scratch_shapes=[
            pltpu.VMEM((2, DFF, D), jnp.float32),
            pltpu.VMEM((2, D, DFF), jnp.float32),
            pltpu.VMEM((DFF, D), jnp.bfloat16),
            pltpu.VMEM((D, DFF), jnp.bfloat16),
            pltpu.SemaphoreType.DMA((2, 2)),
            pltpu.SMEM((1,), jnp.int32),
        ],
    )
    return pl.pallas_call(
        _gmm_body,
        grid_spec=grid_spec,
        out_shape=jax.ShapeDtypeStruct((NS, D), jnp.float32),
    )(be, xs, w1, b1.reshape(E, 1, DFF), w2, b2.reshape(E, 1, D),
      ws.reshape(NS, 1))


def _combine_body(y_hbm, pos0_hbm, pos1_hbm, out_hbm,
                  p0v, p1v, buf0, buf1, sem):
    cid = lax.axis_index("c")
    sid = lax.axis_index("s")
    wid = sid * 2 + cid
    base = wid * TOK_PER_TILE

    pltpu.sync_copy(pos0_hbm.at[pl.ds(base, TOK_PER_TILE)], p0v)
    pltpu.sync_copy(pos1_hbm.at[pl.ds(base, TOK_PER_TILE)], p1v)
    c0 = pltpu.async_copy(y_hbm.at[p0v], buf0, sem)
    c1 = pltpu.async_copy(y_hbm.at[p1v], buf1, sem)
    c0.wait()
    c1.wait()

    def tbody(t, _):
        def jbody(j, _):
            s = pl.ds(j * 16, 16)
            buf0[t, s] = buf0[t, s] + buf1[t, s]
            return 0
        return lax.fori_loop(0, D // 16, jbody, 0)

    lax.fori_loop(0, TOK_PER_TILE, tbody, 0)
    pltpu.sync_copy(buf0, out_hbm.at[pl.ds(base, TOK_PER_TILE)])


def _combine(y, pos0, pos1):
    mesh = plsc.VectorSubcoreMesh(core_axis_name="c", subcore_axis_name="s")
    f = functools.partial(
        pl.kernel,
        out_type=jax.ShapeDtypeStruct((T, D), jnp.float32),
        mesh=mesh,
        scratch_types=[
            pltpu.VMEM((TOK_PER_TILE,), jnp.int32),
            pltpu.VMEM((TOK_PER_TILE,), jnp.int32),
            pltpu.VMEM((TOK_PER_TILE, D), jnp.float32),
            pltpu.VMEM((TOK_PER_TILE, D), jnp.float32),
            pltpu.SemaphoreType.DMA,
        ],
    )(_combine_body)
    return f(y, pos0, pos1)


def kernel(x, Wg, W1, b1, W2, b2):
    B, S, d = x.shape
    x_flat = x.reshape(T, D)
    p0, p1, w0, w1, blk = _router(x_flat, Wg)
    pos0 = p0.reshape(T)
    pos1 = p1.reshape(T)
    be = blk[:, 0] + 0
    xs, ws = _dispatch(pos0, pos1, w0.reshape(T), w1.reshape(T), x_flat)
    y = _gmm(be, xs, ws, W1, b1, W2, b2)
    out = _combine(y, pos0, pos1)
    return out.reshape(B, S, D), 0.0


# speedup vs baseline: 1.2282x; 1.0056x over previous
import functools

import jax
import jax.numpy as jnp
from jax import lax
from jax.experimental import pallas as pl
from jax.experimental.pallas import tpu as pltpu
from jax.experimental.pallas import tpu_sc as plsc

T = 2048
D = 768
DFF = 3072
E = 8
BM = 256
NB = 24
NS = NB * BM
NWORK = 32
ROWS_PER_TILE = NS // NWORK
TOK_PER_TILE = T // NWORK


def _router_body(x_ref, wg_ref, p0_ref, p1_ref, w0_ref, w1_ref, blk_ref):
    x = x_ref[...]
    wg = wg_ref[...]
    logits = lax.dot_general(x, wg, (((1,), (1,)), ((), ())),
                             preferred_element_type=jnp.float32)
    eio = lax.broadcasted_iota(jnp.int32, (T, E), 1)
    m0 = jnp.max(logits, axis=1, keepdims=True)
    idx0 = jnp.min(jnp.where(logits == m0, eio, E), axis=1, keepdims=True)
    lmask = jnp.where(eio == idx0, -jnp.inf, logits)
    m1 = jnp.max(lmask, axis=1, keepdims=True)
    idx1 = jnp.min(jnp.where(lmask == m1, eio, E), axis=1, keepdims=True)
    p1 = jnp.exp(m1 - m0)
    w0 = 1.0 / (1.0 + p1)
    w1 = p1 / (1.0 + p1)

    oh0 = (eio == idx0).astype(jnp.float32)
    oh1 = (eio == idx1).astype(jnp.float32)
    cnt = oh0 + oh1

    prior = jnp.zeros((T, E), jnp.float32)
    tio = lax.broadcasted_iota(jnp.int32, (T, BM), 0)
    cio = lax.broadcasted_iota(jnp.int32, (T, BM), 1)
    for s in range(T // BM):
        ls = (cio + s * BM < tio).astype(jnp.float32)
        cs = cnt[s * BM:(s + 1) * BM, :]
        prior = prior + lax.dot_general(
            ls, cs, (((1,), (0,)), ((), ())),
            preferred_element_type=jnp.float32)

    totals = jnp.sum(cnt, axis=0, keepdims=True)
    pc = (((totals.astype(jnp.int32) + BM - 1) // BM) * BM).astype(jnp.float32)
    l8a = lax.broadcasted_iota(jnp.int32, (E, E), 0)
    l8b = lax.broadcasted_iota(jnp.int32, (E, E), 1)
    l8 = (l8a < l8b).astype(jnp.float32)
    start = lax.dot_general(pc, l8, (((1,), (0,)), ((), ())),
                            preferred_element_type=jnp.float32)

    base = start + prior
    p0_ref[...] = jnp.sum(jnp.where(eio == idx0, base, 0.0),
                          axis=1, keepdims=True).astype(jnp.int32)
    p1_ref[...] = jnp.sum(jnp.where(eio == idx1, base, 0.0),
                          axis=1, keepdims=True).astype(jnp.int32)
    w0_ref[...] = w0
    w1_ref[...] = w1

    endblk = ((start + pc) * (1.0 / BM)).astype(jnp.int32)
    bio = lax.broadcasted_iota(jnp.int32, (NB, E), 0)
    ge = (bio >= jnp.broadcast_to(endblk, (NB, E))).astype(jnp.int32)
    bexp = jnp.minimum(jnp.sum(ge, axis=1, keepdims=True), E - 1)
    blk_ref[...] = jnp.broadcast_to(bexp, (NB, E))


def _router(x_flat, wg):
    return pl.pallas_call(
        _router_body,
        out_shape=(
            jax.ShapeDtypeStruct((T, 1), jnp.int32),
            jax.ShapeDtypeStruct((T, 1), jnp.int32),
            jax.ShapeDtypeStruct((T, 1), jnp.float32),
            jax.ShapeDtypeStruct((T, 1), jnp.float32),
            jax.ShapeDtypeStruct((NB, E), jnp.int32),
        ),
    )(x_flat, wg)


def _dispatch_body(pos0_hbm, pos1_hbm, w0_hbm, w1_hbm, x_hbm,
                   xs_hbm, ws_hbm,
                   pos0_v, pos1_v, w0_v, w1_v, sp0_v, sp1_v, row_v, sw_v,
                   w_sh, sem, wsem):
    cid = lax.axis_index("c")
    sid = lax.axis_index("s")
    wid = sid * 2 + cid
    tbase = wid * TOK_PER_TILE

    pltpu.sync_copy(pos0_hbm.at[pl.ds(tbase, TOK_PER_TILE)], pos0_v)
    pltpu.sync_copy(pos1_hbm.at[pl.ds(tbase, TOK_PER_TILE)], pos1_v)
    pltpu.sync_copy(x_hbm.at[pl.ds(tbase, TOK_PER_TILE)], row_v)
    c0 = pltpu.async_copy(row_v, xs_hbm.at[pos0_v], sem)
    c1 = pltpu.async_copy(row_v, xs_hbm.at[pos1_v], sem)

    sbase = sid * (T // 16)
    pltpu.sync_copy(pos0_hbm.at[pl.ds(sbase, T // 16)], sp0_v)
    pltpu.sync_copy(pos1_hbm.at[pl.ds(sbase, T // 16)], sp1_v)
    pltpu.sync_copy(w0_hbm.at[pl.ds(sbase, T // 16)], w0_v)
    pltpu.sync_copy(w1_hbm.at[pl.ds(sbase, T // 16)], w1_v)
    pltpu.sync_copy(w0_v, w_sh.at[sp0_v])
    pltpu.sync_copy(w1_v, w_sh.at[sp1_v])
    plsc.subcore_barrier()
    pltpu.sync_copy(w_sh.at[pl.ds(wid * ROWS_PER_TILE, ROWS_PER_TILE)], sw_v)
    c2 = pltpu.async_copy(sw_v, ws_hbm.at[pl.ds(wid * ROWS_PER_TILE,
                                                ROWS_PER_TILE)], wsem)
    c0.wait()
    c1.wait()
    c2.wait()


def _dispatch(pos0, pos1, w0, w1, x_flat):
    mesh = plsc.VectorSubcoreMesh(core_axis_name="c", subcore_axis_name="s")
    f = functools.partial(
        pl.kernel,
        out_type=(jax.ShapeDtypeStruct((NS, D), jnp.float32),
                  jax.ShapeDtypeStruct((NS,), jnp.float32)),
        mesh=mesh,
        scratch_types=[
            pltpu.VMEM((TOK_PER_TILE,), jnp.int32),
            pltpu.VMEM((TOK_PER_TILE,), jnp.int32),
            pltpu.VMEM((T // 16,), jnp.float32),
            pltpu.VMEM((T // 16,), jnp.float32),
            pltpu.VMEM((T // 16,), jnp.int32),
            pltpu.VMEM((T // 16,), jnp.int32),
            pltpu.VMEM((TOK_PER_TILE, D), jnp.float32),
            pltpu.VMEM((ROWS_PER_TILE,), jnp.float32),
            pltpu.VMEM_SHARED((NS,), jnp.float32),
            pltpu.SemaphoreType.DMA,
            pltpu.SemaphoreType.DMA,
        ],
    )(_dispatch_body)
    return f(pos0, pos1, w0, w1, x_flat)


def _wconv_body(w1_ref, w2_ref, o1_ref, o2_ref):
    o1_ref[...] = w1_ref[...].astype(jnp.bfloat16)
    o2_ref[...] = w2_ref[...].astype(jnp.bfloat16)


def _wconv(w1, w2):
    return pl.pallas_call(
        _wconv_body,
        grid=(E, 2),
        in_specs=[
            pl.BlockSpec((1, DFF // 2, D), lambda e, i: (e, i, 0)),
            pl.BlockSpec((1, D // 2, DFF), lambda e, i: (e, i, 0)),
        ],
        out_specs=[
            pl.BlockSpec((1, DFF // 2, D), lambda e, i: (e, i, 0)),
            pl.BlockSpec((1, D // 2, DFF), lambda e, i: (e, i, 0)),
        ],
        out_shape=(jax.ShapeDtypeStruct((E, DFF, D), jnp.bfloat16),
                   jax.ShapeDtypeStruct((E, D, DFF), jnp.bfloat16)),
    )(w1, w2)


def _gmm_body(be_ref, x_ref, w1_hbm, b1_ref, w2_hbm, b2_ref, ws_ref, o_ref,
              w1_raw, w2_raw, w1_bf, w2_bf, sems, slot_ref):
    b = pl.program_id(0)
    e = be_ref[b]

    def start_copies(expert, slot):
        pltpu.make_async_copy(w1_hbm.at[expert], w1_raw.at[slot],
                              sems.at[slot, 0]).start()
        pltpu.make_async_copy(w2_hbm.at[expert], w2_raw.at[slot],
                              sems.at[slot, 1]).start()

    @pl.when(b == 0)
    def _():
        slot_ref[0] = 0
        start_copies(e, 0)

    nxt = jnp.minimum(b + 1, NB - 1)
    changes = (b + 1 < NB) & (be_ref[nxt] != e)

    @pl.when(changes)
    def _():
        start_copies(be_ref[nxt], slot_ref[0] ^ 1)

    prv = jnp.maximum(b - 1, 0)

    @pl.when((b == 0) | (be_ref[prv] != e))
    def _():
        s = slot_ref[0]
        pltpu.make_async_copy(w1_hbm.at[e], w1_raw.at[s],
                              sems.at[s, 0]).wait()
        pltpu.make_async_copy(w2_hbm.at[e], w2_raw.at[s],
                              sems.at[s, 1]).wait()
        w1_bf[...] = w1_raw[s].astype(jnp.bfloat16)
        w2_bf[...] = w2_raw[s].astype(jnp.bfloat16)

    x = x_ref[...].astype(jnp.bfloat16)
    h = lax.dot_general(x, w1_bf[...], (((1,), (1,)), ((), ())),
                        preferred_element_type=jnp.float32)
    h = h + b1_ref[0]
    h = 0.5 * h * (1.0 + lax.erf(h * 0.7071067811865476))
    o = lax.dot_general(h.astype(jnp.bfloat16), w2_bf[...],
                        (((1,), (1,)), ((), ())),
                        preferred_element_type=jnp.float32)
    o_ref[...] = (o + b2_ref[0]) * ws_ref[...]

    @pl.when(changes)
    def _():
        slot_ref[0] = slot_ref[0] ^ 1


def _gmm(be, xs, ws, w1, b1, w2, b2):
    grid_spec = pltpu.PrefetchScalarGridSpec(
        num_scalar_prefetch=1,
        grid=(NB,),
        in_specs=[
            pl.BlockSpec((BM, D), lambda b, be: (b, 0)),
            pl.BlockSpec(memory_space=pl.ANY),
            pl.BlockSpec((1, 1, DFF), lambda b, be: (be[b], 0, 0)),
            pl.BlockSpec(memory_space=pl.ANY),
            pl.BlockSpec((1, 1, D), lambda b, be: (be[b], 0, 0)),
            pl.BlockSpec((BM, 1), lambda b, be: (b, 0)),
        ],
        out_specs=pl.BlockSpec((BM, D), lambda b, be: (b, 0)),
        scratch_shapes=[
            pltpu.VMEM((2, DFF, D), jnp.float32),
            pltpu.VMEM((2, D, DFF), jnp.float32),
            pltpu.VMEM((DFF, D), jnp.bfloat16),
            pltpu.VMEM((D, DFF), jnp.bfloat16),
            pltpu.SemaphoreType.DMA((2, 2)),
            pltpu.SMEM((1,), jnp.int32),
        ],
    )
    return pl.pallas_call(
        _gmm_body,
        grid_spec=grid_spec,
        out_shape=jax.ShapeDtypeStruct((NS, D), jnp.float32),
    )(be, xs, w1, b1.reshape(E, 1, DFF), w2, b2.reshape(E, 1, D),
      ws.reshape(NS, 1))


def _combine_body(y_hbm, pos0_hbm, pos1_hbm, out_hbm,
                  p0v, p1v, buf0, buf1, gsems, ssem):
    cid = lax.axis_index("c")
    sid = lax.axis_index("s")
    wid = sid * 2 + cid
    base = wid * TOK_PER_TILE
    nch = 4
    ck = TOK_PER_TILE // nch

    pltpu.sync_copy(pos0_hbm.at[pl.ds(base, TOK_PER_TILE)], p0v)
    pltpu.sync_copy(pos1_hbm.at[pl.ds(base, TOK_PER_TILE)], p1v)
    for c in range(nch):
        pltpu.make_async_copy(y_hbm.at[p0v.at[pl.ds(c * ck, ck)]],
                              buf0.at[pl.ds(c * ck, ck)],
                              gsems.at[c]).start()
        pltpu.make_async_copy(y_hbm.at[p1v.at[pl.ds(c * ck, ck)]],
                              buf1.at[pl.ds(c * ck, ck)],
                              gsems.at[c]).start()
    for c in range(nch):
        pltpu.make_async_copy(y_hbm.at[p0v.at[pl.ds(c * ck, ck)]],
                              buf0.at[pl.ds(c * ck, ck)],
                              gsems.at[c]).wait()
        pltpu.make_async_copy(y_hbm.at[p1v.at[pl.ds(c * ck, ck)]],
                              buf1.at[pl.ds(c * ck, ck)],
                              gsems.at[c]).wait()

        def tbody(t, _):
            def jbody(j, _):
                s = pl.ds(j * 16, 16)
                buf0[t, s] = buf0[t, s] + buf1[t, s]
                return 0
            return lax.fori_loop(0, D // 16, jbody, 0)

        lax.fori_loop(c * ck, (c + 1) * ck, tbody, 0)
        pltpu.make_async_copy(buf0.at[pl.ds(c * ck, ck)],
                              out_hbm.at[pl.ds(base + c * ck, ck)],
                              ssem).start()
    for c in range(nch):
        pltpu.make_async_copy(buf0.at[pl.ds(c * ck, ck)],
                              out_hbm.at[pl.ds(base + c * ck, ck)],
                              ssem).wait()


def _combine(y, pos0, pos1):
    mesh = plsc.VectorSubcoreMesh(core_axis_name="c", subcore_axis_name="s")
    f = functools.partial(
        pl.kernel,
        out_type=jax.ShapeDtypeStruct((T, D), jnp.float32),
        mesh=mesh,
        scratch_types=[
            pltpu.VMEM((TOK_PER_TILE,), jnp.int32),
            pltpu.VMEM((TOK_PER_TILE,), jnp.int32),
            pltpu.VMEM((TOK_PER_TILE, D), jnp.float32),
            pltpu.VMEM((TOK_PER_TILE, D), jnp.float32),
            pltpu.SemaphoreType.DMA((4,)),
            pltpu.SemaphoreType.DMA,
        ],
    )(_combine_body)
    return f(y, pos0, pos1)


def kernel(x, Wg, W1, b1, W2, b2):
    B, S, d = x.shape
    x_flat = x.reshape(T, D)
    p0, p1, w0, w1, blk = _router(x_flat, Wg)
    pos0 = p0.reshape(T)
    pos1 = p1.reshape(T)
    be = blk[:, 0] + 0
    xs, ws = _dispatch(pos0, pos1, w0.reshape(T), w1.reshape(T), x_flat)
    y = _gmm(be, xs, ws, W1, b1, W2, b2)
    out = _combine(y, pos0, pos1)
    return out.reshape(B, S, D), 0.0


# speedup vs baseline: 1.2771x; 1.0399x over previous
import functools

import jax
import jax.numpy as jnp
from jax import lax
from jax.experimental import pallas as pl
from jax.experimental.pallas import tpu as pltpu
from jax.experimental.pallas import tpu_sc as plsc

T = 2048
D = 768
DFF = 3072
E = 8
BM = 256
NB = 24
NS = NB * BM
NWORK = 32
ROWS_PER_TILE = NS // NWORK
TOK_PER_TILE = T // NWORK


def _router_body(x_ref, wg_ref, p0_ref, p1_ref, w0_ref, w1_ref, blk_ref):
    x = x_ref[...]
    wg = wg_ref[...]
    logits = lax.dot_general(x, wg, (((1,), (1,)), ((), ())),
                             preferred_element_type=jnp.float32)
    eio = lax.broadcasted_iota(jnp.int32, (T, E), 1)
    m0 = jnp.max(logits, axis=1, keepdims=True)
    idx0 = jnp.min(jnp.where(logits == m0, eio, E), axis=1, keepdims=True)
    lmask = jnp.where(eio == idx0, -jnp.inf, logits)
    m1 = jnp.max(lmask, axis=1, keepdims=True)
    idx1 = jnp.min(jnp.where(lmask == m1, eio, E), axis=1, keepdims=True)
    p1 = jnp.exp(m1 - m0)
    w0 = 1.0 / (1.0 + p1)
    w1 = p1 / (1.0 + p1)

    oh0 = (eio == idx0).astype(jnp.float32)
    oh1 = (eio == idx1).astype(jnp.float32)
    cnt = oh0 + oh1

    prior = jnp.zeros((T, E), jnp.float32)
    tio = lax.broadcasted_iota(jnp.int32, (T, BM), 0)
    cio = lax.broadcasted_iota(jnp.int32, (T, BM), 1)
    for s in range(T // BM):
        ls = (cio + s * BM < tio).astype(jnp.float32)
        cs = cnt[s * BM:(s + 1) * BM, :]
        prior = prior + lax.dot_general(
            ls, cs, (((1,), (0,)), ((), ())),
            preferred_element_type=jnp.float32)

    totals = jnp.sum(cnt, axis=0, keepdims=True)
    pc = (((totals.astype(jnp.int32) + BM - 1) // BM) * BM).astype(jnp.float32)
    l8a = lax.broadcasted_iota(jnp.int32, (E, E), 0)
    l8b = lax.broadcasted_iota(jnp.int32, (E, E), 1)
    l8 = (l8a < l8b).astype(jnp.float32)
    start = lax.dot_general(pc, l8, (((1,), (0,)), ((), ())),
                            preferred_element_type=jnp.float32)

    base = start + prior
    p0_ref[...] = jnp.sum(jnp.where(eio == idx0, base, 0.0),
                          axis=1, keepdims=True).astype(jnp.int32)
    p1_ref[...] = jnp.sum(jnp.where(eio == idx1, base, 0.0),
                          axis=1, keepdims=True).astype(jnp.int32)
    w0_ref[...] = w0
    w1_ref[...] = w1

    endblk = ((start + pc) * (1.0 / BM)).astype(jnp.int32)
    bio = lax.broadcasted_iota(jnp.int32, (NB, E), 0)
    ge = (bio >= jnp.broadcast_to(endblk, (NB, E))).astype(jnp.int32)
    bexp = jnp.minimum(jnp.sum(ge, axis=1, keepdims=True), E - 1)
    blk_ref[...] = jnp.broadcast_to(bexp, (NB, E))


def _router(x_flat, wg):
    return pl.pallas_call(
        _router_body,
        out_shape=(
            jax.ShapeDtypeStruct((T, 1), jnp.int32),
            jax.ShapeDtypeStruct((T, 1), jnp.int32),
            jax.ShapeDtypeStruct((T, 1), jnp.float32),
            jax.ShapeDtypeStruct((T, 1), jnp.float32),
            jax.ShapeDtypeStruct((NB, E), jnp.int32),
        ),
    )(x_flat, wg)


def _dispatch_body(pos0_hbm, pos1_hbm, x_hbm, xs_hbm,
                   pos0_v, pos1_v, row_v, sem):
    cid = lax.axis_index("c")
    sid = lax.axis_index("s")
    wid = sid * 2 + cid
    tbase = wid * TOK_PER_TILE

    pltpu.sync_copy(pos0_hbm.at[pl.ds(tbase, TOK_PER_TILE)], pos0_v)
    pltpu.sync_copy(pos1_hbm.at[pl.ds(tbase, TOK_PER_TILE)], pos1_v)
    pltpu.sync_copy(x_hbm.at[pl.ds(tbase, TOK_PER_TILE)], row_v)
    c0 = pltpu.async_copy(row_v, xs_hbm.at[pos0_v], sem)
    c1 = pltpu.async_copy(row_v, xs_hbm.at[pos1_v], sem)
    c0.wait()
    c1.wait()


def _dispatch(pos0, pos1, x_flat):
    mesh = plsc.VectorSubcoreMesh(core_axis_name="c", subcore_axis_name="s")
    f = functools.partial(
        pl.kernel,
        out_type=jax.ShapeDtypeStruct((NS, D), jnp.float32),
        mesh=mesh,
        scratch_types=[
            pltpu.VMEM((TOK_PER_TILE,), jnp.int32),
            pltpu.VMEM((TOK_PER_TILE,), jnp.int32),
            pltpu.VMEM((TOK_PER_TILE, D), jnp.float32),
            pltpu.SemaphoreType.DMA,
        ],
    )(_dispatch_body)
    return f(pos0, pos1, x_flat)


def _wconv_body(w1_ref, w2_ref, o1_ref, o2_ref):
    o1_ref[...] = w1_ref[...].astype(jnp.bfloat16)
    o2_ref[...] = w2_ref[...].astype(jnp.bfloat16)


def _wconv(w1, w2):
    return pl.pallas_call(
        _wconv_body,
        grid=(E, 2),
        in_specs=[
            pl.BlockSpec((1, DFF // 2, D), lambda e, i: (e, i, 0)),
            pl.BlockSpec((1, D // 2, DFF), lambda e, i: (e, i, 0)),
        ],
        out_specs=[
            pl.BlockSpec((1, DFF // 2, D), lambda e, i: (e, i, 0)),
            pl.BlockSpec((1, D // 2, DFF), lambda e, i: (e, i, 0)),
        ],
        out_shape=(jax.ShapeDtypeStruct((E, DFF, D), jnp.bfloat16),
                   jax.ShapeDtypeStruct((E, D, DFF), jnp.bfloat16)),
    )(w1, w2)


def _gmm_body(be_ref, x_ref, w1_hbm, b1_ref, w2_hbm, b2_ref, o_ref,
              w1_raw, w2_raw, w1_bf, w2_bf, sems, slot_ref):
    b = pl.program_id(0)
    e = be_ref[b]

    def start_copies(expert, slot):
        pltpu.make_async_copy(w1_hbm.at[expert], w1_raw.at[slot],
                              sems.at[slot, 0]).start()
        pltpu.make_async_copy(w2_hbm.at[expert], w2_raw.at[slot],
                              sems.at[slot, 1]).start()

    @pl.when(b == 0)
    def _():
        slot_ref[0] = 0
        start_copies(e, 0)

    nxt = jnp.minimum(b + 1, NB - 1)
    changes = (b + 1 < NB) & (be_ref[nxt] != e)

    @pl.when(changes)
    def _():
        start_copies(be_ref[nxt], slot_ref[0] ^ 1)

    prv = jnp.maximum(b - 1, 0)

    @pl.when((b == 0) | (be_ref[prv] != e))
    def _():
        s = slot_ref[0]
        pltpu.make_async_copy(w1_hbm.at[e], w1_raw.at[s],
                              sems.at[s, 0]).wait()
        pltpu.make_async_copy(w2_hbm.at[e], w2_raw.at[s],
                              sems.at[s, 1]).wait()
        w1_bf[...] = w1_raw[s].astype(jnp.bfloat16)
        w2_bf[...] = w2_raw[s].astype(jnp.bfloat16)

    x = x_ref[...].astype(jnp.bfloat16)
    h = lax.dot_general(x, w1_bf[...], (((1,), (1,)), ((), ())),
                        preferred_element_type=jnp.float32)
    h = h + b1_ref[0]
    h = 0.5 * h * (1.0 + lax.erf(h * 0.7071067811865476))
    o = lax.dot_general(h.astype(jnp.bfloat16), w2_bf[...],
                        (((1,), (1,)), ((), ())),
                        preferred_element_type=jnp.float32)
    o_ref[...] = o + b2_ref[0]

    @pl.when(changes)
    def _():
        slot_ref[0] = slot_ref[0] ^ 1


def _gmm(be, xs, w1, b1, w2, b2):
    grid_spec = pltpu.PrefetchScalarGridSpec(
        num_scalar_prefetch=1,
        grid=(NB,),
        in_specs=[
            pl.BlockSpec((BM, D), lambda b, be: (b, 0)),
            pl.BlockSpec(memory_space=pl.ANY),
            pl.BlockSpec((1, 1, DFF), lambda b, be: (be[b], 0, 0)),
            pl.BlockSpec(memory_space=pl.ANY),
            pl.BlockSpec((1, 1, D), lambda b, be: (be[b], 0, 0)),
        ],
        out_specs=pl.BlockSpec((BM, D), lambda b, be: (b, 0)),
        scratch_shapes=[
            pltpu.VMEM((2, DFF, D), jnp.float32),
            pltpu.VMEM((2, D, DFF), jnp.float32),
            pltpu.VMEM((DFF, D), jnp.bfloat16),
            pltpu.VMEM((D, DFF), jnp.bfloat16),
            pltpu.SemaphoreType.DMA((2, 2)),
            pltpu.SMEM((1,), jnp.int32),
        ],
    )
    return pl.pallas_call(
        _gmm_body,
        grid_spec=grid_spec,
        out_shape=jax.ShapeDtypeStruct((NS, D), jnp.float32),
    )(be, xs, w1, b1.reshape(E, 1, DFF), w2, b2.reshape(E, 1, D))


def _combine_body(y_hbm, pos0_hbm, pos1_hbm, w0_hbm, w1_hbm, out_hbm,
                  p0v, p1v, w0v, w1v, buf0, buf1, gsems, ssem):
    cid = lax.axis_index("c")
    sid = lax.axis_index("s")
    wid = sid * 2 + cid
    base = wid * TOK_PER_TILE
    nch = 4
    ck = TOK_PER_TILE // nch

    pltpu.sync_copy(pos0_hbm.at[pl.ds(base, TOK_PER_TILE)], p0v)
    pltpu.sync_copy(pos1_hbm.at[pl.ds(base, TOK_PER_TILE)], p1v)
    pltpu.sync_copy(w0_hbm.at[pl.ds(base, TOK_PER_TILE)], w0v)
    pltpu.sync_copy(w1_hbm.at[pl.ds(base, TOK_PER_TILE)], w1v)
    for c in range(nch):
        pltpu.make_async_copy(y_hbm.at[p0v.at[pl.ds(c * ck, ck)]],
                              buf0.at[pl.ds(c * ck, ck)],
                              gsems.at[c]).start()
        pltpu.make_async_copy(y_hbm.at[p1v.at[pl.ds(c * ck, ck)]],
                              buf1.at[pl.ds(c * ck, ck)],
                              gsems.at[c]).start()
    for c in range(nch):
        pltpu.make_async_copy(y_hbm.at[p0v.at[pl.ds(c * ck, ck)]],
                              buf0.at[pl.ds(c * ck, ck)],
                              gsems.at[c]).wait()
        pltpu.make_async_copy(y_hbm.at[p1v.at[pl.ds(c * ck, ck)]],
                              buf1.at[pl.ds(c * ck, ck)],
                              gsems.at[c]).wait()

        wc0 = w0v[pl.ds(c * ck, ck)]
        wc1 = w1v[pl.ds(c * ck, ck)]

        def tbody(t, _):
            lane = jax.lax.broadcast(t - c * ck, (16,))
            w0s = jax.lax.gather(
                wc0, lane[:, None],
                jax.lax.GatherDimensionNumbers(
                    offset_dims=(), collapsed_slice_dims=(0,),
                    start_index_map=(0,)),
                (1,), mode=jax.lax.GatherScatterMode.PROMISE_IN_BOUNDS)
            w1s = jax.lax.gather(
                wc1, lane[:, None],
                jax.lax.GatherDimensionNumbers(
                    offset_dims=(), collapsed_slice_dims=(0,),
                    start_index_map=(0,)),
                (1,), mode=jax.lax.GatherScatterMode.PROMISE_IN_BOUNDS)

            def jbody(j, _):
                s = pl.ds(j * 16, 16)
                buf0[t, s] = w0s * buf0[t, s] + w1s * buf1[t, s]
                return 0
            return lax.fori_loop(0, D // 16, jbody, 0)

        lax.fori_loop(c * ck, (c + 1) * ck, tbody, 0)
        pltpu.make_async_copy(buf0.at[pl.ds(c * ck, ck)],
                              out_hbm.at[pl.ds(base + c * ck, ck)],
                              ssem).start()
    for c in range(nch):
        pltpu.make_async_copy(buf0.at[pl.ds(c * ck, ck)],
                              out_hbm.at[pl.ds(base + c * ck, ck)],
                              ssem).wait()


def _combine(y, pos0, pos1, w0, w1):
    mesh = plsc.VectorSubcoreMesh(core_axis_name="c", subcore_axis_name="s")
    f = functools.partial(
        pl.kernel,
        out_type=jax.ShapeDtypeStruct((T, D), jnp.float32),
        mesh=mesh,
        scratch_types=[
            pltpu.VMEM((TOK_PER_TILE,), jnp.int32),
            pltpu.VMEM((TOK_PER_TILE,), jnp.int32),
            pltpu.VMEM((TOK_PER_TILE,), jnp.float32),
            pltpu.VMEM((TOK_PER_TILE,), jnp.float32),
            pltpu.VMEM((TOK_PER_TILE, D), jnp.float32),
            pltpu.VMEM((TOK_PER_TILE, D), jnp.float32),
            pltpu.SemaphoreType.DMA((4,)),
            pltpu.SemaphoreType.DMA,
        ],
    )(_combine_body)
    return f(y, pos0, pos1, w0, w1)


def kernel(x, Wg, W1, b1, W2, b2):
    B, S, d = x.shape
    x_flat = x.reshape(T, D)
    p0, p1, w0, w1, blk = _router(x_flat, Wg)
    pos0 = p0.reshape(T)
    pos1 = p1.reshape(T)
    be = blk[:, 0] + 0
    xs = _dispatch(pos0, pos1, x_flat)
    y = _gmm(be, xs, W1, b1, W2, b2)
    out = _combine(y, pos0, pos1, w0.reshape(T), w1.reshape(T))
    return out.reshape(B, S, D), 0.0
